# baseline (speedup 1.0000x reference)
import functools

import jax
import jax.numpy as jnp
from jax import lax
from jax.experimental import pallas as pl
from jax.experimental.pallas import tpu as pltpu
from jax.experimental.pallas import tpu_sc as plsc

_N, _N1, _N2 = 50000, 10000, 2048
_D_IN, _D_H, _D_OUT = 128, 128, 41
_E1, _E2 = 320000, 65536

_NC, _NS = 2, 16
_NW = _NC * _NS
_D1 = _D_H + 16
_D2 = 80


def _make_seg_sum(S, D, LSUB, NT, BT, NBUF):
    assert NT % BT == 0 and BT >= NBUF >= 2
    assert LSUB % 8 == 0 and LSUB <= 128
    SP = S // _NS
    assert (D * 4) % 64 == 0
    zchunks = [LSUB] * (SP // LSUB)
    if SP % LSUB:
        zchunks.append(SP % LSUB)

    mesh = plsc.VectorSubcoreMesh(core_axis_name="c", subcore_axis_name="s")

    @functools.partial(
        pl.kernel,
        out_type=jax.ShapeDtypeStruct((_NC, S, D), jnp.float32),
        mesh=mesh,
        compiler_params=pltpu.CompilerParams(use_tc_tiling_on_sc=False),
        scratch_types=[
            pltpu.VMEM((BT * LSUB,), jnp.int32),
            pltpu.VMEM((BT * LSUB,), jnp.int32),
            [pltpu.VMEM((LSUB, D), jnp.float32) for _ in range(NBUF)],
            pltpu.VMEM_SHARED((S, D), jnp.float32),
            [pltpu.SemaphoreType.DMA for _ in range(NBUF)],
            [pltpu.SemaphoreType.DMA for _ in range(NBUF)],
        ],
    )
    def seg(src_hbm, dst_hbm, table_hbm, acc_out,
            idx_s, idx_d, rows, acc_sh, gsem, ssem):
        c = lax.axis_index("c")
        s = lax.axis_index("s")
        wid = c * _NS + s

        def fill_zb(i, carry):
            r = i // (D // 16)
            col = (i % (D // 16)) * 16
            rows[0][r, pl.ds(col, 16)] = jnp.zeros((16,), jnp.float32)
            return carry
        lax.fori_loop(0, LSUB * (D // 16), fill_zb, 0)

        row0 = s * SP
        zoff = 0
        pend = []
        for i, zc in enumerate(zchunks):
            pend.append(pltpu.async_copy(
                rows[0].at[pl.ds(0, zc)],
                acc_sh.at[pl.ds(row0 + zoff, zc)], ssem[i % NBUF]))
            zoff += zc
        for cp in pend:
            cp.wait()
        plsc.subcore_barrier()

        ebase = wid * (NT * LSUB)

        def batch(k, carry):
            pltpu.sync_copy(src_hbm.at[pl.ds(ebase + k * (BT * LSUB),
                                             BT * LSUB)], idx_s)
            pltpu.sync_copy(dst_hbm.at[pl.ds(ebase + k * (BT * LSUB),
                                             BT * LSUB)], idx_d)
            pend_s = [None] * NBUF
            pend_g = [None] * NBUF
            for j in range(BT + NBUF - 1):
                if j < BT:
                    b = j % NBUF
                    if pend_s[b] is not None:
                        pend_s[b].wait()
                        pend_s[b] = None
                    pend_g[b] = pltpu.async_copy(
                        table_hbm.at[idx_s.at[pl.ds(j * LSUB, LSUB)]], rows[b],
                        gsem[b])
                jj = j - (NBUF - 1)
                if jj >= 0:
                    bb = jj % NBUF
                    pend_g[bb].wait()
                    pend_s[bb] = pltpu.async_copy(
                        rows[bb], acc_sh.at[idx_d.at[pl.ds(jj * LSUB, LSUB)]],
                        ssem[bb],
                        add=True)
            for p in pend_s:
                if p is not None:
                    p.wait()
            return carry
        lax.fori_loop(0, NT // BT, batch, 0)
        plsc.subcore_barrier()

        zoff = 0
        pend = []
        for i, zc in enumerate(zchunks):
            pend.append(pltpu.async_copy(
                acc_sh.at[pl.ds(row0 + zoff, zc)],
                acc_out.at[c, pl.ds(row0 + zoff, zc)], gsem[i % NBUF]))
            zoff += zc
        for cp in pend:
            cp.wait()

    return seg


_seg1 = _make_seg_sum(_N1, _D1, LSUB=80, NT=125, BT=5, NBUF=3)
_seg2 = _make_seg_sum(_N2, _D2, LSUB=128, NT=16, BT=4, NBUF=3)

_DN = (((1,), (1,)), ((), ()))


def _tc_mid_body(acc_ref, x_ref, wl1_ref, bl1_ref, wr1_ref, wl2_ref,
                 h_ref, z_ref):
    B = x_ref.shape[0]
    summed = acc_ref[0] + acc_ref[1]
    cnt = summed[:, _D_H]
    mean = summed[:, :_D_H] / jnp.maximum(cnt, 1.0)[:, None]
    t = lax.dot_general(mean, wl1_ref[...], _DN,
                        preferred_element_type=jnp.float32)
    t = t + bl1_ref[...]
    t = t + lax.dot_general(x_ref[...], wr1_ref[...], _DN,
                            preferred_element_type=jnp.float32)
    h = jnp.maximum(t, 0.0)
    h_ref[...] = h
    z = lax.dot_general(h, wl2_ref[...], _DN,
                        preferred_element_type=jnp.float32)
    z_ref[...] = jnp.concatenate(
        [z, jnp.zeros((B, 64 - _D_OUT), jnp.float32),
         jnp.ones((B, 16), jnp.float32)], axis=1)


def _tc_mid(acc, x1, Wl1, bl1, Wr1, Wl2):
    B = 1000
    return pl.pallas_call(
        _tc_mid_body,
        grid=(_N1 // B,),
        in_specs=[
            pl.BlockSpec((2, B, _D1), lambda i: (0, i, 0)),
            pl.BlockSpec((B, _D_IN), lambda i: (i, 0)),
            pl.BlockSpec((_D_H, _D_IN), lambda i: (0, 0)),
            pl.BlockSpec((1, _D_H), lambda i: (0, 0)),
            pl.BlockSpec((_D_H, _D_IN), lambda i: (0, 0)),
            pl.BlockSpec((_D_OUT, _D_H), lambda i: (0, 0)),
        ],
        out_specs=[
            pl.BlockSpec((B, _D_H), lambda i: (i, 0)),
            pl.BlockSpec((B, _D2), lambda i: (i, 0)),
        ],
        out_shape=[
            jax.ShapeDtypeStruct((_N1, _D_H), jnp.float32),
            jax.ShapeDtypeStruct((_N1, _D2), jnp.float32),
        ],
    )(acc, x1, Wl1, bl1, Wr1, Wl2)


def _tc_final_body(acc_ref, h_ref, wr2_ref, bl2_ref, out_ref):
    summed = acc_ref[0] + acc_ref[1]
    cnt = summed[:, 64]
    mean = summed[:, :_D_OUT] / jnp.maximum(cnt, 1.0)[:, None]
    t = mean + lax.dot_general(h_ref[...], wr2_ref[...], _DN,
                               preferred_element_type=jnp.float32)
    t = t + bl2_ref[...]
    m = jnp.max(t, axis=1, keepdims=True)
    e = t - m
    lse = jnp.log(jnp.sum(jnp.exp(e), axis=1, keepdims=True))
    out_ref[...] = e - lse


def _tc_final(acc, h2, Wr2, bl2):
    return pl.pallas_call(
        _tc_final_body,
        out_shape=jax.ShapeDtypeStruct((_N2, _D_OUT), jnp.float32),
    )(acc, h2, Wr2, bl2)


def kernel(x, edge_index1, edge_index2, Wl1, bl1, Wr1, Wl2, bl2, Wr2):
    x1 = x[:_N1]
    x1aug = jnp.concatenate([x1, jnp.ones((_N1, 16), jnp.float32)], axis=1)
    acc1 = _seg1(edge_index1[0], edge_index1[1], x1aug)
    h, z2 = _tc_mid(acc1, x1, Wl1, bl1.reshape(1, -1), Wr1, Wl2)
    acc2 = _seg2(edge_index2[0], edge_index2[1], z2)
    return _tc_final(acc2, h[:_N2], Wr2, bl2.reshape(1, -1))


# speedup vs baseline: 1.1648x; 1.1648x over previous
import functools

import jax
import jax.numpy as jnp
from jax import lax
from jax.experimental import pallas as pl
from jax.experimental.pallas import tpu as pltpu
from jax.experimental.pallas import tpu_sc as plsc

_N, _N1, _N2 = 50000, 10000, 2048
_D_IN, _D_H, _D_OUT = 128, 128, 41
_E1, _E2 = 320000, 65536

_NC, _NS = 2, 16
_NW = _NC * _NS
_S1 = 10240
_D2 = 80


def _make_seg_sum(S, D, LSUB, NT, BT, NBUF, with_counts=False):
    assert NT % BT == 0 and BT >= NBUF >= 2
    assert LSUB % 8 == 0 and LSUB <= 128
    SP = S // _NS
    assert (D * 4) % 64 == 0
    zchunks = [LSUB] * (SP // LSUB)
    if SP % LSUB:
        zchunks.append(SP % LSUB)

    mesh = plsc.VectorSubcoreMesh(core_axis_name="c", subcore_axis_name="s")

    out_type = jax.ShapeDtypeStruct((_NC, S, D), jnp.float32)
    if with_counts:
        out_type = (out_type, jax.ShapeDtypeStruct((_NW, S), jnp.float32))

    @functools.partial(
        pl.kernel,
        out_type=out_type,
        mesh=mesh,
        compiler_params=pltpu.CompilerParams(
            use_tc_tiling_on_sc=False, needs_layout_passes=False),
        scratch_types=[
            pltpu.VMEM((BT * LSUB,), jnp.int32),
            pltpu.VMEM((BT * LSUB,), jnp.int32),
            [pltpu.VMEM((LSUB, D), jnp.float32) for _ in range(NBUF)],
            pltpu.VMEM((S if with_counts else 16,), jnp.float32),
            pltpu.VMEM_SHARED((S, D), jnp.float32),
            [pltpu.SemaphoreType.DMA for _ in range(NBUF)],
            [pltpu.SemaphoreType.DMA for _ in range(NBUF)],
        ],
    )
    def seg(src_hbm, dst_hbm, table_hbm, *rest):
        if with_counts:
            (acc_out, cnt_out, idx_s, idx_d, rows, cnt_v, acc_sh,
             gsem, ssem) = rest
        else:
            (acc_out, idx_s, idx_d, rows, cnt_v, acc_sh,
             gsem, ssem) = rest
        c = lax.axis_index("c")
        s = lax.axis_index("s")
        wid = c * _NS + s

        if with_counts:
            def zc_body(i, carry):
                cnt_v[pl.ds(i * 16, 16)] = jnp.zeros((16,), jnp.float32)
                return carry
            lax.fori_loop(0, S // 16, zc_body, 0)

        def fill_zb(i, carry):
            r = i // (D // 16)
            col = (i % (D // 16)) * 16
            rows[0][r, pl.ds(col, 16)] = jnp.zeros((16,), jnp.float32)
            return carry
        lax.fori_loop(0, LSUB * (D // 16), fill_zb, 0)

        row0 = s * SP
        zoff = 0
        pend = []
        for i, zc in enumerate(zchunks):
            pend.append(pltpu.async_copy(
                rows[0].at[pl.ds(0, zc)],
                acc_sh.at[pl.ds(row0 + zoff, zc)], ssem[i % NBUF]))
            zoff += zc
        for cp in pend:
            cp.wait()
        plsc.subcore_barrier()

        ebase = wid * (NT * LSUB)

        def batch(k, carry):
            pltpu.sync_copy(src_hbm.at[pl.ds(ebase + k * (BT * LSUB),
                                             BT * LSUB)], idx_s)
            pltpu.sync_copy(dst_hbm.at[pl.ds(ebase + k * (BT * LSUB),
                                             BT * LSUB)], idx_d)
            if with_counts:
                ones16 = jnp.full((16,), 1.0, jnp.float32)

                def cnt_body(i, carry):
                    v = idx_d[pl.ds(i * 16, 16)]
                    plsc.addupdate_scatter(cnt_v, [v], ones16)
                    return carry
                lax.fori_loop(0, (BT * LSUB) // 16, cnt_body, 0)
            pend_s = [None] * NBUF
            pend_g = [None] * NBUF
            for j in range(BT + NBUF - 1):
                if j < BT:
                    b = j % NBUF
                    if pend_s[b] is not None:
                        pend_s[b].wait()
                        pend_s[b] = None
                    pend_g[b] = pltpu.async_copy(
                        table_hbm.at[idx_s.at[pl.ds(j * LSUB, LSUB)]], rows[b],
                        gsem[b])
                jj = j - (NBUF - 1)
                if jj >= 0:
                    bb = jj % NBUF
                    pend_g[bb].wait()
                    pend_s[bb] = pltpu.async_copy(
                        rows[bb], acc_sh.at[idx_d.at[pl.ds(jj * LSUB, LSUB)]],
                        ssem[bb],
                        add=True)
            for p in pend_s:
                if p is not None:
                    p.wait()
            return carry
        lax.fori_loop(0, NT // BT, batch, 0)
        plsc.subcore_barrier()

        zoff = 0
        pend = []
        for i, zc in enumerate(zchunks):
            pend.append(pltpu.async_copy(
                acc_sh.at[pl.ds(row0 + zoff, zc)],
                acc_out.at[c, pl.ds(row0 + zoff, zc)], gsem[i % NBUF]))
            zoff += zc
        if with_counts:
            pend.append(pltpu.async_copy(cnt_v, cnt_out.at[wid],
                                         ssem[0]))
        for cp in pend:
            cp.wait()

    return seg


_seg1 = _make_seg_sum(_S1, _D_H, LSUB=80, NT=125, BT=5, NBUF=3,
                      with_counts=True)
_seg2 = _make_seg_sum(_N2, _D2, LSUB=128, NT=16, BT=4, NBUF=3)

_DN = (((1,), (1,)), ((), ()))


def _tc_mid_body(acc_ref, cnt_ref, x_ref, wl1_ref, bl1_ref, wr1_ref,
                 wl2_ref, h_ref, z_ref):
    B = x_ref.shape[0]
    summed = acc_ref[0] + acc_ref[1]
    cnt = jnp.sum(cnt_ref[...], axis=0)
    mean = summed / jnp.maximum(cnt, 1.0)[:, None]
    t = lax.dot_general(mean, wl1_ref[...], _DN,
                        preferred_element_type=jnp.float32)
    t = t + bl1_ref[...]
    t = t + lax.dot_general(x_ref[...], wr1_ref[...], _DN,
                            preferred_element_type=jnp.float32)
    h = jnp.maximum(t, 0.0)
    h_ref[...] = h
    z = lax.dot_general(h, wl2_ref[...], _DN,
                        preferred_element_type=jnp.float32)
    z_ref[...] = jnp.concatenate(
        [z, jnp.zeros((B, 64 - _D_OUT), jnp.float32),
         jnp.ones((B, 16), jnp.float32)], axis=1)


def _tc_mid(acc, cnt, x1, Wl1, bl1, Wr1, Wl2):
    B = 1024
    return pl.pallas_call(
        _tc_mid_body,
        grid=(_S1 // B,),
        in_specs=[
            pl.BlockSpec((2, B, _D_H), lambda i: (0, i, 0)),
            pl.BlockSpec((_NW, B), lambda i: (0, i)),
            pl.BlockSpec((B, _D_IN), lambda i: (i, 0)),
            pl.BlockSpec((_D_H, _D_IN), lambda i: (0, 0)),
            pl.BlockSpec((1, _D_H), lambda i: (0, 0)),
            pl.BlockSpec((_D_H, _D_IN), lambda i: (0, 0)),
            pl.BlockSpec((_D_OUT, _D_H), lambda i: (0, 0)),
        ],
        out_specs=[
            pl.BlockSpec((B, _D_H), lambda i: (i, 0)),
            pl.BlockSpec((B, _D2), lambda i: (i, 0)),
        ],
        out_shape=[
            jax.ShapeDtypeStruct((_S1, _D_H), jnp.float32),
            jax.ShapeDtypeStruct((_S1, _D2), jnp.float32),
        ],
    )(acc, cnt, x1, Wl1, bl1, Wr1, Wl2)


def _tc_final_body(acc_ref, h_ref, wr2_ref, bl2_ref, out_ref):
    summed = acc_ref[0] + acc_ref[1]
    cnt = summed[:, 64]
    mean = summed[:, :_D_OUT] / jnp.maximum(cnt, 1.0)[:, None]
    t = mean + lax.dot_general(h_ref[...], wr2_ref[...], _DN,
                               preferred_element_type=jnp.float32)
    t = t + bl2_ref[...]
    m = jnp.max(t, axis=1, keepdims=True)
    e = t - m
    lse = jnp.log(jnp.sum(jnp.exp(e), axis=1, keepdims=True))
    out_ref[...] = e - lse


def _tc_final(acc, h2, Wr2, bl2):
    return pl.pallas_call(
        _tc_final_body,
        out_shape=jax.ShapeDtypeStruct((_N2, _D_OUT), jnp.float32),
    )(acc, h2, Wr2, bl2)


def kernel(x, edge_index1, edge_index2, Wl1, bl1, Wr1, Wl2, bl2, Wr2):
    x1p = x[:_S1]
    acc1, cnt1 = _seg1(edge_index1[0], edge_index1[1], x1p)
    h, z2 = _tc_mid(acc1, cnt1, x1p, Wl1, bl1.reshape(1, -1), Wr1, Wl2)
    acc2 = _seg2(edge_index2[0], edge_index2[1], z2)
    return _tc_final(acc2, h[:_N2], Wr2, bl2.reshape(1, -1))
